# baseline (speedup 1.0000x reference)
import functools

import jax
import jax.numpy as jnp
from jax import lax
from jax.experimental import pallas as pl
from jax.experimental.pallas import tpu as pltpu
from jax.experimental.pallas import tpu_sc as plsc

N = 320000
D = 128
M = 128
S = 10000

NB = N // 128
NW = 32
MAXC = 80
NBP = NW * MAXC
DRAIN = 632
NBUF = 3
IDXC = 8


def _sc_segment_sum(x3, ids2, zeros):
    mesh = plsc.VectorSubcoreMesh(core_axis_name="c", subcore_axis_name="s")

    @functools.partial(
        pl.kernel,
        mesh=mesh,
        out_type=(
            jax.ShapeDtypeStruct((S, D), jnp.float32),
            jax.ShapeDtypeStruct((S, D), jnp.float32),
        ),
        scratch_types=[
            pltpu.VMEM_SHARED((S, D), jnp.float32),
            pltpu.VMEM((IDXC, 128), jnp.int32),
        ]
        + [pltpu.VMEM((128, D), jnp.float32) for _ in range(NBUF)]
        + [pltpu.SemaphoreType.DMA for _ in range(NBUF)],
    )
    def segsum(x_hbm, ids_hbm, z_hbm, out0_hbm, out1_hbm, acc, idx_v, *bufs_sems):
        bufs = bufs_sems[:NBUF]
        sems = bufs_sems[NBUF:]
        cid = lax.axis_index("c")
        sid = lax.axis_index("s")
        wid = sid * 2 + cid

        pltpu.sync_copy(z_hbm, bufs[0])
        zbase = sid * 625
        for k in range(4):
            pltpu.sync_copy(bufs[0], acc.at[pl.ds(zbase + k * 128, 128)])
        pltpu.sync_copy(bufs[0].at[pl.ds(0, 113)], acc.at[pl.ds(zbase + 512, 113)])
        plsc.subcore_barrier()

        c0 = wid * MAXC
        cnt = jnp.clip(NB - c0, 0, MAXC)

        def chunk_body(h, carry):
            base = h * IDXC
            cnt_h = jnp.clip(cnt - base, 0, IDXC)

            @pl.when(cnt_h > 0)
            def _():
                pltpu.sync_copy(ids_hbm.at[pl.ds(c0 + base, IDXC)], idx_v)
                for j in range(NBUF):
                    pltpu.async_copy(x_hbm.at[c0 + base + j], bufs[j], sems[j])

                def body(i, carry2):
                    def step(j):
                        def go():
                            pltpu.make_async_copy(x_hbm.at[0], bufs[j], sems[j]).wait()
                            pltpu.sync_copy(bufs[j], acc.at[idx_v.at[i]], add=True)

                            @pl.when(i + NBUF < cnt_h)
                            def _():
                                pltpu.async_copy(
                                    x_hbm.at[c0 + base + i + NBUF], bufs[j], sems[j]
                                )

                        return go

                    jmod = lax.rem(i, NBUF)
                    for j in range(NBUF):
                        pl.when(jmod == j)(step(j))
                    return carry2

                lax.fori_loop(0, cnt_h, body, 0)

            return carry

        lax.fori_loop(0, MAXC // IDXC, chunk_body, 0)
        plsc.subcore_barrier()

        dbase = sid * DRAIN

        def drain_to(out_hbm):
            for k in range(4):
                pltpu.sync_copy(
                    acc.at[pl.ds(dbase + k * 128, 128)],
                    out_hbm.at[pl.ds(dbase + k * 128, 128)],
                )

            @pl.when(sid < 15)
            def _():
                pltpu.sync_copy(
                    acc.at[pl.ds(dbase + 512, 120)],
                    out_hbm.at[pl.ds(dbase + 512, 120)],
                )

            @pl.when(sid == 15)
            def _():
                pltpu.sync_copy(
                    acc.at[pl.ds(dbase + 512, 8)],
                    out_hbm.at[pl.ds(dbase + 512, 8)],
                )

        pl.when(cid == 0)(lambda: drain_to(out0_hbm))
        pl.when(cid == 1)(lambda: drain_to(out1_hbm))

    return segsum(x3, ids2, zeros)


def _tc_matmul(p0, p1, w):
    BS = 2000

    def mm(p0_ref, p1_ref, w_ref, o_ref):
        p = p0_ref[...] + p1_ref[...]
        o_ref[...] = lax.dot_general(
            p, w_ref[...], (((1,), (1,)), ((), ())),
            preferred_element_type=jnp.float32,
        )

    return pl.pallas_call(
        mm,
        grid=(S // BS,),
        in_specs=[
            pl.BlockSpec((BS, D), lambda i: (i, 0)),
            pl.BlockSpec((BS, D), lambda i: (i, 0)),
            pl.BlockSpec((M, D), lambda i: (0, 0)),
        ],
        out_specs=pl.BlockSpec((BS, M), lambda i: (i, 0)),
        out_shape=jax.ShapeDtypeStruct((S, M), jnp.float32),
    )(p0, p1, w)


def kernel(tensor1_values, tensor1_segment_ids, tensor2_values):
    x3 = tensor1_values.reshape(NB, 128, D)
    ids2 = tensor1_segment_ids.astype(jnp.int32).reshape(NB, 128)
    ids2 = jnp.concatenate([ids2, jnp.zeros((NBP - NB, 128), jnp.int32)])
    zeros = jnp.zeros((128, D), jnp.float32)
    p0, p1 = _sc_segment_sum(x3, ids2, zeros)
    return _tc_matmul(p0, p1, tensor2_values)
